# Initial kernel scaffold; baseline (speedup 1.0000x reference)
import jax
import jax.numpy as jnp
from jax.experimental import pallas as pl


def kernel(pred_labels, pred_masks, pred_scores):
    raise NotImplementedError("write your pallas kernel here")


# speedup vs baseline: 4.1370x; 4.1370x over previous
import functools

import jax
import jax.numpy as jnp
from jax.experimental import pallas as pl
from jax.experimental.pallas import tpu as pltpu

Q = 512
S = 20000
C = 18
THR = 0.75
K = 100
TILE = 2048
GRID = (S + TILE - 1) // TILE


def _body(masks_ref, labels_ref, pscore_ref, out_ref, inter_ref, sig_ref, sious_ref):
    j = pl.program_id(0)

    x = masks_ref[...]
    col = jax.lax.broadcasted_iota(jnp.int32, (Q, TILE), 1) + j * TILE
    validpos = (col < S) & (x > 0.0)
    bin16 = jnp.where(validpos, 1.0, 0.0).astype(jnp.bfloat16)
    part = jax.lax.dot_general(
        bin16, bin16, (((1,), (1,)), ((), ())),
        preferred_element_type=jnp.float32)
    sig = jnp.where(validpos, jax.nn.sigmoid(x), 0.0)
    sigpart = jax.lax.dot_general(
        jnp.ones((1, TILE), jnp.float32), sig, (((1,), (1,)), ((), ())),
        precision=jax.lax.Precision.HIGHEST,
        preferred_element_type=jnp.float32)

    @pl.when(j == 0)
    def _():
        inter_ref[...] = part
        sig_ref[...] = sigpart

    @pl.when(j > 0)
    def _():
        inter_ref[...] = inter_ref[...] + part
        sig_ref[...] = sig_ref[...] + sigpart

    @pl.when(j == GRID - 1)
    def _():
        lt = labels_ref[...]
        m = jnp.max(lt, axis=0, keepdims=True)
        e = jnp.exp(lt - m)
        denom = jnp.sum(e, axis=0, keepdims=True)
        scores_t = (e[:C, :] / denom) * pscore_ref[...]

        inter = inter_ref[...]
        ri = jax.lax.broadcasted_iota(jnp.int32, (Q, Q), 0)
        ci = jax.lax.broadcasted_iota(jnp.int32, (Q, Q), 1)
        eye = ri == ci
        pn_row = jnp.sum(jnp.where(eye, inter, 0.0), axis=0, keepdims=True)
        pn_col = jnp.sum(jnp.where(eye, inter, 0.0), axis=1, keepdims=True)

        nms_row = jnp.max(scores_t, axis=0, keepdims=True)
        nms_row = jnp.where(pn_row == 0.0, 0.0, nms_row)

        union = pn_col + pn_row - inter + 1e-6
        iou = inter / union

        s_col = jnp.sum(jnp.where(eye, jnp.broadcast_to(nms_row, (Q, Q)), 0.0),
                        axis=1, keepdims=True)
        gt = jnp.where(nms_row > s_col, 1.0, 0.0)
        eqlt = jnp.where((nms_row == s_col) & (ci < ri), 1.0, 0.0)
        rank_col = jnp.sum(gt + eqlt, axis=1, keepdims=True)
        rank_row = jnp.sum(jnp.where(eye, jnp.broadcast_to(rank_col, (Q, Q)), 0.0),
                           axis=0, keepdims=True)
        perm = jnp.where(ri.astype(jnp.float32) == rank_row, 1.0, 0.0)

        piou = jax.lax.dot_general(
            perm, iou, (((1,), (0,)), ((), ())),
            precision=jax.lax.Precision.HIGHEST,
            preferred_element_type=jnp.float32)
        sious = jax.lax.dot_general(
            piou, perm, (((1,), (1,)), ((), ())),
            precision=jax.lax.Precision.HIGHEST,
            preferred_element_type=jnp.float32)
        sious_ref[...] = sious

        lane = jax.lax.broadcasted_iota(jnp.int32, (1, Q), 1)

        def nms_step(r, keep):
            keep_r = jnp.sum(jnp.where(lane == r, keep, 0.0))
            row = sious_ref[pl.ds(r, 1), :]
            supf = jnp.where((row > THR) & (lane > r), 1.0, 0.0) * keep_r
            return keep * (1.0 - supf)

        keep_s = jax.lax.fori_loop(0, Q, nms_step, jnp.ones((1, Q), jnp.float32))
        keep_row = jax.lax.dot_general(
            keep_s, perm, (((1,), (0,)), ((), ())),
            precision=jax.lax.Precision.HIGHEST,
            preferred_element_type=jnp.float32)

        w_row = sig_ref[...] / (pn_row + 1e-6)
        a = scores_t * keep_row

        fidx = (jax.lax.broadcasted_iota(jnp.int32, (C, Q), 1) * C
                + jax.lax.broadcasted_iota(jnp.int32, (C, Q), 0)).astype(jnp.float32)
        wb = jnp.broadcast_to(w_row, (C, Q))
        lane128 = jax.lax.broadcasted_iota(jnp.int32, (1, 128), 1)

        def topk_step(k, carry):
            a_cur, acc = carry
            v = jnp.max(a_cur)
            pick = jnp.min(jnp.where(a_cur == v, fidx, 1e9))
            onehot = fidx == pick
            wv = jnp.sum(jnp.where(onehot, wb, 0.0))
            acc = jnp.where(lane128 == k, v * wv, acc)
            a_cur = jnp.where(onehot, -1.0, a_cur)
            return a_cur, acc

        _, acc = jax.lax.fori_loop(
            0, K, topk_step, (a, jnp.zeros((1, 128), jnp.float32)))
        out_ref[...] = acc


@functools.partial(jax.jit, static_argnames=())
def kernel(pred_labels, pred_masks, pred_scores):
    labels_t = pred_labels.T
    pscore_row = pred_scores.reshape(1, Q)
    out = pl.pallas_call(
        _body,
        grid=(GRID,),
        in_specs=[
            pl.BlockSpec((Q, TILE), lambda j: (0, j)),
            pl.BlockSpec((C + 1, Q), lambda j: (0, 0)),
            pl.BlockSpec((1, Q), lambda j: (0, 0)),
        ],
        out_specs=pl.BlockSpec((1, 128), lambda j: (0, 0)),
        out_shape=jax.ShapeDtypeStruct((1, 128), jnp.float32),
        scratch_shapes=[
            pltpu.VMEM((Q, Q), jnp.float32),
            pltpu.VMEM((1, Q), jnp.float32),
            pltpu.VMEM((Q, Q), jnp.float32),
        ],
    )(pred_masks, labels_t, pscore_row)
    return out[0, :K]


# speedup vs baseline: 7.7936x; 1.8839x over previous
import functools

import jax
import jax.numpy as jnp
from jax.experimental import pallas as pl
from jax.experimental.pallas import tpu as pltpu

Q = 512
S = 20000
C = 18
THR = 0.75
K = 100
TILE = 2048
GRID = (S + TILE - 1) // TILE


def _body(masks_ref, labels_ref, pscore_ref, out_ref, inter_ref, sig_ref):
    j = pl.program_id(0)

    x = masks_ref[...]
    lane_t = jax.lax.broadcasted_iota(jnp.int32, (1, TILE), 1)
    validrow = jnp.where(lane_t < S - j * TILE, 1.0, 0.0)
    pos = x > 0.0
    bin16 = jnp.where(pos, 1.0, 0.0).astype(jnp.bfloat16) * validrow.astype(jnp.bfloat16)
    part = jax.lax.dot_general(
        bin16, bin16, (((1,), (1,)), ((), ())),
        preferred_element_type=jnp.float32)
    sig = jnp.where(pos, 0.5 + 0.5 * jnp.tanh(0.5 * x), 0.0)
    sigpart = jax.lax.dot_general(
        validrow, sig, (((1,), (1,)), ((), ())),
        precision=jax.lax.Precision.HIGHEST,
        preferred_element_type=jnp.float32)

    @pl.when(j == 0)
    def _():
        inter_ref[...] = part
        sig_ref[...] = sigpart

    @pl.when(j > 0)
    def _():
        inter_ref[...] = inter_ref[...] + part
        sig_ref[...] = sig_ref[...] + sigpart

    @pl.when(j == GRID - 1)
    def _():
        lt = labels_ref[...]
        m = jnp.max(lt, axis=0, keepdims=True)
        e = jnp.exp(lt - m)
        denom = jnp.sum(e, axis=0, keepdims=True)
        scores_t = (e[:C, :] / denom) * pscore_ref[...]

        inter = inter_ref[...]
        ri = jax.lax.broadcasted_iota(jnp.int32, (Q, Q), 0)
        ci = jax.lax.broadcasted_iota(jnp.int32, (Q, Q), 1)
        eye = ri == ci
        pn_row = jnp.sum(jnp.where(eye, inter, 0.0), axis=0, keepdims=True)
        pn_col = jnp.sum(jnp.where(eye, inter, 0.0), axis=1, keepdims=True)

        nms_row = jnp.max(scores_t, axis=0, keepdims=True)
        nms_row = jnp.where(pn_row == 0.0, 0.0, nms_row)

        union = pn_col + pn_row - inter + 1e-6
        adj = jnp.where(inter > THR * union, 1.0, 0.0).astype(jnp.bfloat16)

        s_col = jnp.sum(jnp.where(eye, jnp.broadcast_to(nms_row, (Q, Q)), 0.0),
                        axis=1, keepdims=True)
        gt = jnp.where(nms_row > s_col, 1.0, 0.0)
        eqlt = jnp.where((nms_row == s_col) & (ci < ri), 1.0, 0.0)
        rank_col = jnp.sum(gt + eqlt, axis=1, keepdims=True)
        rank_row = jnp.sum(jnp.where(eye, jnp.broadcast_to(rank_col, (Q, Q)), 0.0),
                           axis=0, keepdims=True)
        perm = jnp.where(ri.astype(jnp.float32) == rank_row, 1.0, 0.0)
        perm16 = perm.astype(jnp.bfloat16)

        padj = jax.lax.dot_general(
            perm16, adj, (((1,), (0,)), ((), ())),
            preferred_element_type=jnp.float32)
        sadj = jax.lax.dot_general(
            padj.astype(jnp.bfloat16), perm16, (((1,), (1,)), ((), ())),
            preferred_element_type=jnp.float32)
        sadj16 = jnp.where(ci > ri, sadj, 0.0).astype(jnp.bfloat16)

        def nms_cond(carry):
            _, changed = carry
            return changed

        def nms_sweep(carry):
            keep, _ = carry
            supcnt = jax.lax.dot_general(
                keep.astype(jnp.bfloat16), sadj16, (((1,), (0,)), ((), ())),
                preferred_element_type=jnp.float32)
            keep_new = jnp.where(supcnt > 0.0, 0.0, 1.0)
            changed = jnp.sum(jnp.abs(keep_new - keep)) > 0.0
            return keep_new, changed

        keep, _ = jax.lax.while_loop(
            nms_cond, nms_sweep, (jnp.ones((1, Q), jnp.float32), True))

        keep_row = jax.lax.dot_general(
            keep.astype(jnp.bfloat16), perm16, (((1,), (0,)), ((), ())),
            preferred_element_type=jnp.float32)

        w_row = sig_ref[...] / (pn_row + 1e-6)
        a = scores_t * keep_row

        wb = jnp.broadcast_to(w_row, (C, Q))
        lane128 = jax.lax.broadcasted_iota(jnp.int32, (1, 128), 1)

        acc = jnp.zeros((1, 128), jnp.float32)
        for k in range(K):
            v = jnp.max(a)
            hit = a == v
            wv = jnp.sum(jnp.where(hit, wb, 0.0))
            acc = jnp.where(lane128 == k, jnp.maximum(v, 0.0) * wv, acc)
            a = jnp.where(hit, -1.0, a)
        out_ref[...] = acc


@functools.partial(jax.jit, static_argnames=())
def kernel(pred_labels, pred_masks, pred_scores):
    labels_t = pred_labels.T
    pscore_row = pred_scores.reshape(1, Q)
    out = pl.pallas_call(
        _body,
        grid=(GRID,),
        in_specs=[
            pl.BlockSpec((Q, TILE), lambda j: (0, j)),
            pl.BlockSpec((C + 1, Q), lambda j: (0, 0)),
            pl.BlockSpec((1, Q), lambda j: (0, 0)),
        ],
        out_specs=pl.BlockSpec((1, 128), lambda j: (0, 0)),
        out_shape=jax.ShapeDtypeStruct((1, 128), jnp.float32),
        scratch_shapes=[
            pltpu.VMEM((Q, Q), jnp.float32),
            pltpu.VMEM((1, Q), jnp.float32),
        ],
    )(pred_masks, labels_t, pscore_row)
    return out[0, :K]


# speedup vs baseline: 7.8162x; 1.0029x over previous
import functools

import jax
import jax.numpy as jnp
from jax.experimental import pallas as pl
from jax.experimental.pallas import tpu as pltpu

Q = 512
S = 20000
C = 18
THR = 0.75
K = 100
TILE = 4096
GRID = (S + TILE - 1) // TILE


def _body(masks_ref, labels_ref, pscore_ref, out_ref, inter_ref, sig_ref):
    j = pl.program_id(0)

    x = masks_ref[...]
    lane_t = jax.lax.broadcasted_iota(jnp.int32, (1, TILE), 1)
    validrow = jnp.where(lane_t < S - j * TILE, 1.0, 0.0)
    pos = x > 0.0
    bin16 = jnp.where(pos, 1.0, 0.0).astype(jnp.bfloat16) * validrow.astype(jnp.bfloat16)
    part = jax.lax.dot_general(
        bin16, bin16, (((1,), (1,)), ((), ())),
        preferred_element_type=jnp.float32)
    sig = jnp.where(pos, 0.5 + 0.5 * jnp.tanh(0.5 * x), 0.0)
    sigpart = jax.lax.dot_general(
        validrow, sig, (((1,), (1,)), ((), ())),
        precision=jax.lax.Precision.HIGHEST,
        preferred_element_type=jnp.float32)

    @pl.when(j == 0)
    def _():
        inter_ref[...] = part
        sig_ref[...] = sigpart

    @pl.when(j > 0)
    def _():
        inter_ref[...] = inter_ref[...] + part
        sig_ref[...] = sig_ref[...] + sigpart

    @pl.when(j == GRID - 1)
    def _():
        lt = labels_ref[...]
        m = jnp.max(lt, axis=0, keepdims=True)
        e = jnp.exp(lt - m)
        denom = jnp.sum(e, axis=0, keepdims=True)
        scores_t = (e[:C, :] / denom) * pscore_ref[...]

        inter = inter_ref[...]
        ri = jax.lax.broadcasted_iota(jnp.int32, (Q, Q), 0)
        ci = jax.lax.broadcasted_iota(jnp.int32, (Q, Q), 1)
        eye = ri == ci
        pn_row = jnp.sum(jnp.where(eye, inter, 0.0), axis=0, keepdims=True)
        pn_col = jnp.sum(jnp.where(eye, inter, 0.0), axis=1, keepdims=True)

        nms_row = jnp.max(scores_t, axis=0, keepdims=True)
        nms_row = jnp.where(pn_row == 0.0, 0.0, nms_row)

        union = pn_col + pn_row - inter + 1e-6
        adj = jnp.where(inter > THR * union, 1.0, 0.0).astype(jnp.bfloat16)

        s_col = jnp.sum(jnp.where(eye, jnp.broadcast_to(nms_row, (Q, Q)), 0.0),
                        axis=1, keepdims=True)
        gt = jnp.where(nms_row > s_col, 1.0, 0.0)
        eqlt = jnp.where((nms_row == s_col) & (ci < ri), 1.0, 0.0)
        rank_col = jnp.sum(gt + eqlt, axis=1, keepdims=True)
        rank_row = jnp.sum(jnp.where(eye, jnp.broadcast_to(rank_col, (Q, Q)), 0.0),
                           axis=0, keepdims=True)
        perm = jnp.where(ri.astype(jnp.float32) == rank_row, 1.0, 0.0)
        perm16 = perm.astype(jnp.bfloat16)

        padj = jax.lax.dot_general(
            perm16, adj, (((1,), (0,)), ((), ())),
            preferred_element_type=jnp.float32)
        sadj = jax.lax.dot_general(
            padj.astype(jnp.bfloat16), perm16, (((1,), (1,)), ((), ())),
            preferred_element_type=jnp.float32)
        sadj16 = jnp.where(ci > ri, sadj, 0.0).astype(jnp.bfloat16)

        def nms_cond(carry):
            _, changed = carry
            return changed

        def nms_sweep(carry):
            keep, _ = carry
            supcnt = jax.lax.dot_general(
                keep.astype(jnp.bfloat16), sadj16, (((1,), (0,)), ((), ())),
                preferred_element_type=jnp.float32)
            keep_new = jnp.where(supcnt > 0.0, 0.0, 1.0)
            changed = jnp.sum(jnp.abs(keep_new - keep)) > 0.0
            return keep_new, changed

        keep, _ = jax.lax.while_loop(
            nms_cond, nms_sweep, (jnp.ones((1, Q), jnp.float32), True))

        keep_row = jax.lax.dot_general(
            keep.astype(jnp.bfloat16), perm16, (((1,), (0,)), ((), ())),
            preferred_element_type=jnp.float32)

        w_row = sig_ref[...] / (pn_row + 1e-6)
        a = scores_t * keep_row

        wb = jnp.broadcast_to(w_row, (C, Q))
        lane128 = jax.lax.broadcasted_iota(jnp.int32, (1, 128), 1)

        acc = jnp.zeros((1, 128), jnp.float32)
        for k in range(K):
            v = jnp.max(a)
            hit = a == v
            wv = jnp.sum(jnp.where(hit, wb, 0.0))
            acc = jnp.where(lane128 == k, jnp.maximum(v, 0.0) * wv, acc)
            a = jnp.where(hit, -1.0, a)
        out_ref[...] = acc


@functools.partial(jax.jit, static_argnames=())
def kernel(pred_labels, pred_masks, pred_scores):
    labels_t = pred_labels.T
    pscore_row = pred_scores.reshape(1, Q)
    out = pl.pallas_call(
        _body,
        grid=(GRID,),
        in_specs=[
            pl.BlockSpec((Q, TILE), lambda j: (0, j)),
            pl.BlockSpec((C + 1, Q), lambda j: (0, 0)),
            pl.BlockSpec((1, Q), lambda j: (0, 0)),
        ],
        out_specs=pl.BlockSpec((1, 128), lambda j: (0, 0)),
        out_shape=jax.ShapeDtypeStruct((1, 128), jnp.float32),
        scratch_shapes=[
            pltpu.VMEM((Q, Q), jnp.float32),
            pltpu.VMEM((1, Q), jnp.float32),
        ],
    )(pred_masks, labels_t, pscore_row)
    return out[0, :K]
